# baseline (speedup 1.0000x reference)
import jax
import jax.numpy as jnp
from jax import lax
from jax.experimental import pallas as pl
from jax.experimental.pallas import tpu as pltpu
from jax.experimental.pallas import tpu_sc as plsc

D = 64
L = 16
NW = 32
SEQ = 200
BATCH = 1024
B_PER_W = BATCH // NW
G_FULL = SEQ // L
TAIL = SEQ - L
SPLIT = 104


def _sc_body(xi_hbm, tt_hbm, table_hbm, seg_hbm, pe_hbm, out_hbm,
             xi_v, tt_v, pidx_v, seg_v, add_v, rows_v, out_v, sem0, sem1):
    wid = lax.axis_index("s") * 2 + lax.axis_index("c")
    b0 = wid * B_PER_W

    pltpu.sync_copy(xi_hbm.at[pl.ds(b0, B_PER_W)], xi_v)
    pltpu.sync_copy(tt_hbm.at[pl.ds(b0, B_PER_W)], tt_v)
    pltpu.sync_copy(pe_hbm.at[pl.ds(0, SEQ)], out_v)
    pltpu.sync_copy(seg_hbm, seg_v)

    group_offs = [g * L for g in range(G_FULL)] + [TAIL]

    seg0_ = [seg_v[0, pl.ds(d * L, L)] for d in range(4)]
    seg1_ = [seg_v[1, pl.ds(d * L, L)] for d in range(4)]

    def build_add(s, _):
        for d in range(4):
            p = out_v[s, pl.ds(d * L, L)]
            add_v[s, pl.ds(d * L, L)] = p + seg0_[d]
            add_v[s, pl.ds(D + d * L, L)] = p + seg1_[d]
        return 0

    lax.fori_loop(0, SEQ, build_add, 0)

    def issue_gather(q, buf, sem):
        pb = buf * SEQ
        for off in group_offs:
            pidx_v[pl.ds(pb + off, L)] = xi_v[q, pl.ds(off, L)]
        cp0 = pltpu.async_copy(
            table_hbm.at[pidx_v.at[pl.ds(pb, SPLIT)]],
            rows_v.at[buf, pl.ds(0, SPLIT)], sem)
        cp1 = pltpu.async_copy(
            table_hbm.at[pidx_v.at[pl.ds(pb + SPLIT, SEQ - SPLIT)]],
            rows_v.at[buf, pl.ds(SPLIT, SEQ - SPLIT)], sem)
        return cp0, cp1

    def wait_gather(q, buf, sem):
        pb = buf * SEQ
        pltpu.make_async_copy(
            table_hbm.at[pidx_v.at[pl.ds(pb, SPLIT)]],
            rows_v.at[buf, pl.ds(0, SPLIT)], sem).wait()
        pltpu.make_async_copy(
            table_hbm.at[pidx_v.at[pl.ds(pb + SPLIT, SEQ - SPLIT)]],
            rows_v.at[buf, pl.ds(SPLIT, SEQ - SPLIT)], sem).wait()

    issue_gather(0, 0, sem0)

    def row_body(q, _):
        buf = q & 1

        @pl.when(jnp.logical_and(q + 1 < B_PER_W, buf == 0))
        def _():
            issue_gather(q + 1, 1, sem1)

        @pl.when(jnp.logical_and(q + 1 < B_PER_W, buf == 1))
        def _():
            issue_gather(q + 1, 0, sem0)

        @pl.when(buf == 0)
        def _():
            wait_gather(q, 0, sem0)

        @pl.when(buf == 1)
        def _():
            wait_gather(q, 1, sem1)

        for off in group_offs:
            tvec = tt_v[q, pl.ds(off, L)]
            for r16 in range(L):
                r = off + r16
                toff = tvec[r16] * D
                for d in range(4):
                    out_v[r, pl.ds(d * L, L)] = (
                        rows_v[buf, r, pl.ds(d * L, L)]
                        + add_v[r, pl.ds(toff + d * L, L)])

        pltpu.sync_copy(out_v, out_hbm.at[b0 + q])
        return 0

    lax.fori_loop(0, B_PER_W, row_body, 0)


@jax.jit
def _encode(xi, tt, table, segment_table, pe2d):
    mesh = plsc.VectorSubcoreMesh(
        core_axis_name="c", subcore_axis_name="s", num_cores=2, num_subcores=16)
    run = pl.kernel(
        _sc_body,
        out_type=jax.ShapeDtypeStruct((BATCH, SEQ, D), jnp.float32),
        mesh=mesh,
        scratch_types=[
            pltpu.VMEM((B_PER_W, SEQ), jnp.int32),
            pltpu.VMEM((B_PER_W, SEQ), jnp.int32),
            pltpu.VMEM((2 * SEQ,), jnp.int32),
            pltpu.VMEM((8, D), jnp.float32),
            pltpu.VMEM((SEQ, 2 * D), jnp.float32),
            pltpu.VMEM((2, SEQ, 2 * D), jnp.float32),
            pltpu.VMEM((SEQ, D), jnp.float32),
            pltpu.SemaphoreType.DMA,
            pltpu.SemaphoreType.DMA,
        ],
    )
    return run(xi, tt, table, segment_table, pe2d)


def kernel(x, token_types, token_table, segment_table, pe):
    xi = x.astype(jnp.int32)
    tt = token_types.astype(jnp.int32)
    pe2d = pe.reshape(pe.shape[-2], D)
    tablep = jnp.pad(token_table, ((0, 0), (0, D)))
    seg8 = jnp.pad(segment_table, ((0, 6), (0, 0)))
    return _encode(xi, tt, tablep, seg8, pe2d)
